# baseline (speedup 1.0000x reference)
import jax
import jax.numpy as jnp
from jax.experimental import pallas as pl
from jax.experimental.pallas import tpu as pltpu

_TAU = 0.5
_BLOCK_T = 512


def _gate_block_kernel(x_ref, w_ref, out_ref):
    x = x_ref[...].astype(jnp.bfloat16)
    w = w_ref[...].astype(jnp.bfloat16)
    logits = jax.lax.dot_general(
        x, w, (((1,), (1,)), ((), ())), preferred_element_type=jnp.float32
    )
    m = jnp.max(logits, axis=-1, keepdims=True)
    thr = m + jnp.log(jnp.float32(_TAU))
    out_ref[...] = jnp.where(logits < thr, 0.0, 1.0).astype(out_ref.dtype)


def kernel(routing_inputs, W):
    tokens, hidden = routing_inputs.shape
    experts = W.shape[0]
    grid = (tokens // _BLOCK_T,)
    return pl.pallas_call(
        _gate_block_kernel,
        grid=grid,
        in_specs=[
            pl.BlockSpec((_BLOCK_T, hidden), lambda i: (i, 0)),
            pl.BlockSpec((experts, hidden), lambda i: (0, 0)),
        ],
        out_specs=pl.BlockSpec((_BLOCK_T, experts), lambda i: (i, 0)),
        out_shape=jax.ShapeDtypeStruct((tokens, experts), jnp.float32),
        compiler_params=pltpu.CompilerParams(
            dimension_semantics=("arbitrary",),
        ),
    )(routing_inputs, W)


# speedup vs baseline: 1.1680x; 1.1680x over previous
import jax
import jax.numpy as jnp
from jax.experimental import pallas as pl
from jax.experimental.pallas import tpu as pltpu

_TAU = 0.5
_BLOCK_T = 1024
_HALF = _BLOCK_T // 2


def _gate_block_kernel(xa_ref, xb_ref, w_ref, out_ref):
    w = w_ref[...].astype(jnp.bfloat16)
    log_tau = jnp.log(jnp.float32(_TAU))
    for x_ref, off in ((xa_ref, 0), (xb_ref, _HALF)):
        x = x_ref[...].astype(jnp.bfloat16)
        logits = jax.lax.dot_general(
            x, w, (((1,), (1,)), ((), ())), preferred_element_type=jnp.float32
        )
        m = jnp.max(logits, axis=-1, keepdims=True)
        out_ref[pl.ds(off, _HALF), :] = jnp.where(logits < m + log_tau, 0.0, 1.0)


def kernel(routing_inputs, W):
    tokens, hidden = routing_inputs.shape
    experts = W.shape[0]
    grid = (tokens // _BLOCK_T,)
    return pl.pallas_call(
        _gate_block_kernel,
        grid=grid,
        in_specs=[
            pl.BlockSpec((_HALF, hidden), lambda i: (2 * i, 0)),
            pl.BlockSpec((_HALF, hidden), lambda i: (2 * i + 1, 0)),
            pl.BlockSpec((experts, hidden), lambda i: (0, 0)),
        ],
        out_specs=pl.BlockSpec((_BLOCK_T, experts), lambda i: (i, 0)),
        out_shape=jax.ShapeDtypeStruct((tokens, experts), jnp.float32),
        compiler_params=pltpu.CompilerParams(
            dimension_semantics=("arbitrary",),
        ),
    )(routing_inputs, routing_inputs, W)


# speedup vs baseline: 1.1684x; 1.0004x over previous
import jax
import jax.numpy as jnp
from jax.experimental import pallas as pl
from jax.experimental.pallas import tpu as pltpu

_TAU = 0.5
_BLOCK_T = 1024
_HALF = _BLOCK_T // 2


def _gate_block_kernel(xa_ref, xb_ref, w_ref, out_ref):
    w = w_ref[...].astype(jnp.bfloat16)
    log_tau = jnp.log(jnp.float32(_TAU))
    for x_ref, off in ((xa_ref, 0), (xb_ref, _HALF)):
        x = x_ref[...].astype(jnp.bfloat16)
        logits = jax.lax.dot_general(
            x, w, (((1,), (1,)), ((), ())), preferred_element_type=jnp.float32
        )
        m = jnp.max(logits, axis=-1, keepdims=True)
        out_ref[pl.ds(off, _HALF), :] = jnp.where(logits < m + log_tau, 0.0, 1.0)


def kernel(routing_inputs, W):
    tokens, hidden = routing_inputs.shape
    experts = W.shape[0]
    grid = (tokens // _BLOCK_T,)
    return pl.pallas_call(
        _gate_block_kernel,
        grid=grid,
        in_specs=[
            pl.BlockSpec((_HALF, hidden), lambda i: (2 * i, 0)),
            pl.BlockSpec((_HALF, hidden), lambda i: (2 * i + 1, 0)),
            pl.BlockSpec((experts, hidden), lambda i: (0, 0)),
        ],
        out_specs=pl.BlockSpec((_BLOCK_T, experts), lambda i: (i, 0)),
        out_shape=jax.ShapeDtypeStruct((tokens, experts), jnp.float32),
        compiler_params=pltpu.CompilerParams(
            dimension_semantics=("arbitrary",),
            disable_bounds_checks=True,
            skip_device_barrier=True,
        ),
    )(routing_inputs, routing_inputs, W)


# speedup vs baseline: 1.1685x; 1.0001x over previous
import jax
import jax.numpy as jnp
from jax.experimental import pallas as pl
from jax.experimental.pallas import tpu as pltpu

_TAU = 0.5
_BLOCK_T = 1024
_HALF = _BLOCK_T // 2


def _gate_block_kernel(xa_ref, xb_ref, w_ref, out_ref):
    w = w_ref[...].astype(jnp.bfloat16)
    log_tau = jnp.log(jnp.float32(_TAU))
    for x_ref, off in ((xa_ref, 0), (xb_ref, _HALF)):
        x = x_ref[...].astype(jnp.bfloat16)
        logits = jax.lax.dot_general(
            x, w, (((1,), (1,)), ((), ())), preferred_element_type=jnp.float32
        )
        m = jnp.max(logits, axis=-1, keepdims=True)
        out_ref[pl.ds(off, _HALF), :] = jnp.where(logits < m + log_tau, 0.0, 1.0)


def kernel(routing_inputs, W):
    tokens, hidden = routing_inputs.shape
    experts = W.shape[0]
    grid = (tokens // _BLOCK_T,)
    return pl.pallas_call(
        _gate_block_kernel,
        grid=grid,
        in_specs=[
            pl.BlockSpec((_HALF, hidden), lambda i: (2 * i, 0)),
            pl.BlockSpec((_HALF, hidden), lambda i: (2 * i + 1, 0)),
            pl.BlockSpec((experts, hidden), lambda i: (0, 0)),
        ],
        out_specs=pl.BlockSpec((_BLOCK_T, experts), lambda i: (i, 0)),
        out_shape=jax.ShapeDtypeStruct((tokens, experts), jnp.float32),
        compiler_params=pltpu.CompilerParams(
            dimension_semantics=("arbitrary",),
        ),
    )(routing_inputs, routing_inputs, W)
